# Initial kernel scaffold; baseline (speedup 1.0000x reference)
import jax
import jax.numpy as jnp
from jax.experimental import pallas as pl


def kernel(x, W, b, centroids):
    raise NotImplementedError("write your pallas kernel here")


# speedup vs baseline: 1.2771x; 1.2771x over previous
import jax
import jax.numpy as jnp
from jax.experimental import pallas as pl
from jax.experimental.pallas import tpu as pltpu

_B, _N, _D, _K = 32, 8192, 256, 64
_BN = 2048
_NB = _N // _BN


def _netvlad_kernel(x_ref, wt_ref, b_ref, c_ref, out_ref, agg_ref, asum_ref):
    n_idx = pl.program_id(1)

    @pl.when(n_idx == 0)
    def _init():
        agg_ref[...] = jnp.zeros_like(agg_ref)
        asum_ref[...] = jnp.zeros_like(asum_ref)

    x = x_ref[0]
    logits = jax.lax.dot_general(
        x, wt_ref[...], (((1,), (0,)), ((), ())),
        preferred_element_type=jnp.float32,
    ) + b_ref[...]
    m = jnp.max(logits, axis=-1, keepdims=True)
    e = jnp.exp(logits - m)
    s = jnp.sum(e, axis=-1, keepdims=True)
    a = e / s
    agg_ref[...] += jax.lax.dot_general(
        a, x, (((0,), (0,)), ((), ())),
        preferred_element_type=jnp.float32,
    )
    asum_ref[...] += jnp.sum(a, axis=0, keepdims=True)

    @pl.when(n_idx == _NB - 1)
    def _finalize():
        asum_col = asum_ref[...].reshape(_K, 1)
        vlad = agg_ref[...] - asum_col * c_ref[...]
        norm = jnp.sqrt(jnp.sum(vlad * vlad, axis=-1, keepdims=True))
        out_ref[0] = vlad / jnp.maximum(norm, 1e-12)


def kernel(x, W, b, centroids):
    Wt = W.T
    b2 = b.reshape(1, _K)
    out = pl.pallas_call(
        _netvlad_kernel,
        grid=(_B, _NB),
        in_specs=[
            pl.BlockSpec((1, _BN, _D), lambda i, j: (i, j, 0)),
            pl.BlockSpec((_D, _K), lambda i, j: (0, 0)),
            pl.BlockSpec((1, _K), lambda i, j: (0, 0)),
            pl.BlockSpec((_K, _D), lambda i, j: (0, 0)),
        ],
        out_specs=pl.BlockSpec((1, _K, _D), lambda i, j: (i, 0, 0)),
        out_shape=jax.ShapeDtypeStruct((_B, _K, _D), jnp.float32),
        scratch_shapes=[
            pltpu.VMEM((_K, _D), jnp.float32),
            pltpu.VMEM((1, _K), jnp.float32),
        ],
        compiler_params=pltpu.CompilerParams(
            dimension_semantics=("parallel", "arbitrary"),
        ),
    )(x, Wt, b2, centroids)
    return out.reshape(_B, _K * _D)


# speedup vs baseline: 1.6130x; 1.2630x over previous
import jax
import jax.numpy as jnp
from jax.experimental import pallas as pl
from jax.experimental.pallas import tpu as pltpu

_B, _N, _D, _K = 32, 8192, 256, 64
_BN = 4096
_NB = _N // _BN


def _netvlad_kernel(x_ref, wt_ref, b_ref, c_ref, out_ref, agg_ref, asum_ref):
    n_idx = pl.program_id(1)

    @pl.when(n_idx == 0)
    def _init():
        agg_ref[...] = jnp.zeros_like(agg_ref)
        asum_ref[...] = jnp.zeros_like(asum_ref)

    x = x_ref[0]
    logits = jax.lax.dot_general(
        x, wt_ref[...], (((1,), (0,)), ((), ())),
        preferred_element_type=jnp.float32,
    ) + b_ref[...]
    m = jnp.max(logits, axis=-1, keepdims=True)
    e = jnp.exp(logits - m)
    s = jnp.sum(e, axis=-1, keepdims=True)
    a = e / s
    agg_ref[...] += jax.lax.dot_general(
        a, x, (((0,), (0,)), ((), ())),
        preferred_element_type=jnp.float32,
    )
    asum_ref[...] += jnp.sum(a, axis=0, keepdims=True)

    @pl.when(n_idx == _NB - 1)
    def _finalize():
        asum_col = asum_ref[...].reshape(_K, 1)
        vlad = agg_ref[...] - asum_col * c_ref[...]
        norm = jnp.sqrt(jnp.sum(vlad * vlad, axis=-1, keepdims=True))
        out_ref[0] = vlad / jnp.maximum(norm, 1e-12)


def kernel(x, W, b, centroids):
    Wt = W.T
    b2 = b.reshape(1, _K)
    out = pl.pallas_call(
        _netvlad_kernel,
        grid=(_B, _NB),
        in_specs=[
            pl.BlockSpec((1, _BN, _D), lambda i, j: (i, j, 0)),
            pl.BlockSpec((_D, _K), lambda i, j: (0, 0)),
            pl.BlockSpec((1, _K), lambda i, j: (0, 0)),
            pl.BlockSpec((_K, _D), lambda i, j: (0, 0)),
        ],
        out_specs=pl.BlockSpec((1, _K, _D), lambda i, j: (i, 0, 0)),
        out_shape=jax.ShapeDtypeStruct((_B, _K, _D), jnp.float32),
        scratch_shapes=[
            pltpu.VMEM((_K, _D), jnp.float32),
            pltpu.VMEM((1, _K), jnp.float32),
        ],
        compiler_params=pltpu.CompilerParams(
            dimension_semantics=("arbitrary", "arbitrary"),
        ),
    )(x, Wt, b2, centroids)
    return out.reshape(_B, _K * _D)


# speedup vs baseline: 1.9958x; 1.2373x over previous
import jax
import jax.numpy as jnp
from jax.experimental import pallas as pl
from jax.experimental.pallas import tpu as pltpu

_B, _N, _D, _K = 32, 8192, 256, 64
_BN = 8192
_NB = _N // _BN


def _netvlad_kernel(x_ref, wt_ref, b_ref, c_ref, out_ref, agg_ref, asum_ref):
    n_idx = pl.program_id(1)

    @pl.when(n_idx == 0)
    def _init():
        agg_ref[...] = jnp.zeros_like(agg_ref)
        asum_ref[...] = jnp.zeros_like(asum_ref)

    x = x_ref[0]
    logits = jax.lax.dot_general(
        x, wt_ref[...], (((1,), (0,)), ((), ())),
        preferred_element_type=jnp.float32,
    ) + b_ref[...]
    m = jnp.max(logits, axis=-1, keepdims=True)
    e = jnp.exp(logits - m)
    s = jnp.sum(e, axis=-1, keepdims=True)
    a = e / s
    agg_ref[...] += jax.lax.dot_general(
        a, x, (((0,), (0,)), ((), ())),
        preferred_element_type=jnp.float32,
    )
    asum_ref[...] += jnp.sum(a, axis=0, keepdims=True)

    @pl.when(n_idx == _NB - 1)
    def _finalize():
        asum_col = asum_ref[...].reshape(_K, 1)
        vlad = agg_ref[...] - asum_col * c_ref[...]
        norm = jnp.sqrt(jnp.sum(vlad * vlad, axis=-1, keepdims=True))
        out_ref[0] = vlad / jnp.maximum(norm, 1e-12)


def kernel(x, W, b, centroids):
    Wt = W.T
    b2 = b.reshape(1, _K)
    out = pl.pallas_call(
        _netvlad_kernel,
        grid=(_B, _NB),
        in_specs=[
            pl.BlockSpec((1, _BN, _D), lambda i, j: (i, j, 0)),
            pl.BlockSpec((_D, _K), lambda i, j: (0, 0)),
            pl.BlockSpec((1, _K), lambda i, j: (0, 0)),
            pl.BlockSpec((_K, _D), lambda i, j: (0, 0)),
        ],
        out_specs=pl.BlockSpec((1, _K, _D), lambda i, j: (i, 0, 0)),
        out_shape=jax.ShapeDtypeStruct((_B, _K, _D), jnp.float32),
        scratch_shapes=[
            pltpu.VMEM((_K, _D), jnp.float32),
            pltpu.VMEM((1, _K), jnp.float32),
        ],
        compiler_params=pltpu.CompilerParams(
            dimension_semantics=("arbitrary", "arbitrary"),
        ),
    )(x, Wt, b2, centroids)
    return out.reshape(_B, _K * _D)
